# baseline (speedup 1.0000x reference)
import functools

import jax
import jax.numpy as jnp
import numpy as np
from jax import lax
from jax.experimental import pallas as pl
from jax.experimental.pallas import tpu as pltpu
from jax.experimental.pallas import tpu_sc as plsc

N = 4096
B = 16
HID = 128
TED = 128
NL = 5
RADIUS = 1.5
MAXNB = 30
SLOTS = 32
NBASIS = 16
AVG_DEG_LOG = float(np.log(31.0))
RB = 256
NBLK = N // RB
E = N * SLOTS

_vals = np.linspace(0.0, RADIUS, NBASIS + 2)
_STEP = float(_vals[1] - _vals[0])
_CENTERS = np.asarray(_vals[1:-1], dtype=np.float32)


def _mm(a, b):
    return lax.dot(a.astype(jnp.bfloat16), b.astype(jnp.bfloat16),
                   preferred_element_type=jnp.float32)


def _mm_exact(a, b):
    return lax.dot(a, b, precision=lax.Precision.HIGHEST)


def _nbr_body(cxr, cyr, czr, br, cxc, cyc, czc, bc, src_out, elen_out, d_s):
    i = pl.program_id(0)
    dx = cxr[...] - cxc[...]
    dy = cyr[...] - cyc[...]
    dz = czr[...] - czc[...]
    d = jnp.sqrt((dx * dx + dy * dy) + dz * dz)
    rowid = i * RB + lax.broadcasted_iota(jnp.int32, (RB, N), 0)
    colid = lax.broadcasted_iota(jnp.int32, (RB, N), 1)
    invalid = (br[...] != bc[...]) | (rowid == colid)
    d_s[...] = jnp.where(invalid, jnp.inf, d)

    slot = lax.broadcasted_iota(jnp.int32, (RB, SLOTS), 1)

    def body(k, carry):
        vals, idxs = carry
        dcur = d_s[...]
        rowmin = jnp.min(dcur, axis=1, keepdims=True)
        amin = jnp.min(jnp.where(dcur == rowmin, colid, N), axis=1,
                       keepdims=True)
        d_s[...] = jnp.where(colid == amin, jnp.inf, dcur)
        hit = slot == k
        vals = jnp.where(hit, rowmin, vals)
        idxs = jnp.where(hit, amin, idxs)
        return vals, idxs

    vals0 = jnp.full((RB, SLOTS), jnp.inf, jnp.float32)
    idxs0 = jnp.zeros((RB, SLOTS), jnp.int32)
    vals, idxs = lax.fori_loop(0, MAXNB, body, (vals0, idxs0))
    elen_out[...] = vals
    src_out[...] = idxs


def _neighbor_search(coords, batch):
    cxr = coords[:, 0:1]
    cyr = coords[:, 1:2]
    czr = coords[:, 2:3]
    cxc = coords[:, 0].reshape(1, N)
    cyc = coords[:, 1].reshape(1, N)
    czc = coords[:, 2].reshape(1, N)
    br = batch.reshape(N, 1)
    bc = batch.reshape(1, N)
    row_spec = pl.BlockSpec((RB, 1), lambda i: (i, 0))
    col_spec = pl.BlockSpec((1, N), lambda i: (0, 0))
    return pl.pallas_call(
        _nbr_body,
        grid=(NBLK,),
        in_specs=[row_spec, row_spec, row_spec, row_spec,
                  col_spec, col_spec, col_spec, col_spec],
        out_specs=[pl.BlockSpec((RB, SLOTS), lambda i: (i, 0)),
                   pl.BlockSpec((RB, SLOTS), lambda i: (i, 0))],
        out_shape=[jax.ShapeDtypeStruct((N, SLOTS), jnp.int32),
                   jax.ShapeDtypeStruct((N, SLOTS), jnp.float32)],
        scratch_shapes=[pltpu.VMEM((RB, N), jnp.float32)],
    )(cxr, cyr, czr, br, cxc, cyc, czc, bc)


_NW = 32
_BPW = E // _NW
_CH = 256
_NCHUNK = _BPW // _CH


def _sc_gather(table, idx):
    mesh = plsc.VectorSubcoreMesh(core_axis_name="c", subcore_axis_name="s")

    @functools.partial(
        pl.kernel,
        out_type=jax.ShapeDtypeStruct((E, HID), jnp.float32),
        mesh=mesh,
        scratch_types=[
            pltpu.VMEM((_CH,), jnp.int32),
            pltpu.VMEM((_CH,), jnp.int32),
            pltpu.VMEM((_CH, HID), jnp.float32),
            pltpu.VMEM((_CH, HID), jnp.float32),
            pltpu.SemaphoreType.DMA,
            pltpu.SemaphoreType.DMA,
            pltpu.SemaphoreType.DMA,
            pltpu.SemaphoreType.DMA,
        ],
    )
    def gather_k(idx_hbm, table_hbm, out_hbm, iv0, iv1, rv0, rv1,
                 gs0, gs1, ws0, ws1):
        wid = lax.axis_index("s") * 2 + lax.axis_index("c")
        base = wid * _BPW
        ivs = (iv0, iv1)
        rvs = (rv0, rv1)
        gss = (gs0, gs1)
        wss = (ws0, ws1)

        def load_idx(c, b):
            pltpu.sync_copy(idx_hbm.at[pl.ds(base + c * _CH, _CH)], ivs[b])

        def start_gather(b):
            return pltpu.async_copy(table_hbm.at[ivs[b]], rvs[b], gss[b])

        def start_write(c, b):
            return pltpu.async_copy(
                rvs[b], out_hbm.at[pl.ds(base + c * _CH, _CH)], wss[b])

        load_idx(0, 0)
        g = start_gather(0)
        w = [None, None]
        for c in range(_NCHUNK):
            cur = c & 1
            nxt = 1 - cur
            if c + 1 < _NCHUNK:
                load_idx(c + 1, nxt)
            g.wait()
            if c + 1 < _NCHUNK:
                if w[nxt] is not None:
                    w[nxt].wait()
                    w[nxt] = None
                g = start_gather(nxt)
            w[cur] = start_write(c, cur)
        for b in range(2):
            if w[b] is not None:
                w[b].wait()

    return gather_k(idx, table)


def _prologue_body(coords, t, Wf, te_W, te_b, tm_W, tm_b, in_W, in_b, bc,
                   h0, tpn_all, elem):
    xp = (2.0 * jnp.pi) * t[...] * Wf[...]
    tf = jnp.concatenate([jnp.sin(xp), jnp.cos(xp)], axis=-1)
    tf = _mm(tf, te_W[...]) + te_b[...]
    tf = tf * jax.nn.sigmoid(tf)
    for l in range(NL):
        tpn_all[l * B:(l + 1) * B, :] = _mm(tf, tm_W[l]) + tm_b[l]
    h0[...] = _mm(coords[...], in_W[...]) + in_b[...]
    gid = lax.broadcasted_iota(jnp.int32, (B, N), 0)
    cntg = jnp.sum((gid == bc[...]).astype(jnp.float32), axis=1,
                   keepdims=True)
    elem[...] = jnp.maximum(cntg * HID, 1.0)


def _prologue(coords, batch, t, Wf, te_W, te_b, tm_W, tm_b, in_W, in_b):
    return pl.pallas_call(
        _prologue_body,
        out_shape=[jax.ShapeDtypeStruct((N, HID), jnp.float32),
                   jax.ShapeDtypeStruct((NL * B, HID), jnp.float32),
                   jax.ShapeDtypeStruct((B, 1), jnp.float32)],
    )(coords, t.reshape(B, 1), Wf.reshape(1, TED // 2), te_W,
      te_b.reshape(1, TED), tm_W, tm_b, in_W, in_b.reshape(1, HID),
      batch.reshape(1, N))


def _tables_body(h, br, tpn_g, Wd, Ws, pre_b,
                 x_out, yd_out, ys_out):
    oh = (br[...] == lax.broadcasted_iota(jnp.int32, (N, B), 1)).astype(
        jnp.float32)
    x = h[...] + _mm_exact(oh, tpn_g[...])
    x_out[...] = x
    yd_out[...] = _mm(x, Wd[...]) + pre_b[...]
    ys_out[...] = _mm(x, Ws[...])


def _layer_tables(h, batch_r, tpn_g, Wd, Ws, pre_bl):
    return pl.pallas_call(
        _tables_body,
        out_shape=[jax.ShapeDtypeStruct((N, HID), jnp.float32),
                   jax.ShapeDtypeStruct((N, HID), jnp.float32),
                   jax.ShapeDtypeStruct((N, HID), jnp.float32)],
    )(h, batch_r, tpn_g, Wd, Ws, pre_bl.reshape(1, HID))


def _agg_body(x, yd, G, elen, br, bc, ee_W, ee_b, We, Px, P1, P2, P3,
              post_b, lin_W, lin_b, hn_out, gstat_out):
    centers = (lax.broadcasted_iota(jnp.int32, (1, NBASIS), 1).astype(
        jnp.float32) + 1.0) * _STEP
    G3 = G[...].reshape(RB, SLOTS, HID)
    ydv = yd[...]
    s1 = jnp.zeros((RB, HID), jnp.float32)
    s2 = jnp.zeros((RB, HID), jnp.float32)
    mx = jnp.full((RB, HID), -jnp.inf, jnp.float32)
    mn = jnp.full((RB, HID), jnp.inf, jnp.float32)
    cnt = jnp.zeros((RB, 1), jnp.float32)
    for j in range(MAXNB):
        elen_j = elen[:, j:j + 1]
        diff = (elen_j - centers) * (1.0 / _STEP)
        basis = jnp.exp(-(diff * diff)) * (1.0 / 1.12)
        e_j = _mm(basis, ee_W[...]) + ee_b[...]
        m_j = ydv + G3[:, j, :] + _mm(e_j, We[...])
        ok = elen_j <= RADIUS
        w_j = ok.astype(jnp.float32)
        s1 = s1 + w_j * m_j
        s2 = s2 + w_j * (m_j * m_j)
        mx = jnp.maximum(mx, jnp.where(ok, m_j, -jnp.inf))
        mn = jnp.minimum(mn, jnp.where(ok, m_j, jnp.inf))
        cnt = cnt + w_j
    cntc = jnp.maximum(cnt, 1.0)
    inv = 1.0 / cntc
    mean = s1 * inv
    mean2 = s2 * inv
    std = jnp.sqrt(jax.nn.relu(mean2 - mean * mean) + 1e-5)
    has = cnt > 0.0
    mx = jnp.where(has, mx, 0.0)
    mn = jnp.where(has, mn, 0.0)
    agg = jnp.concatenate([mean, mx, mn, std], axis=-1)
    dlog = jnp.log(cntc + 1.0)
    f1 = dlog * (1.0 / AVG_DEG_LOG)
    f2 = AVG_DEG_LOG / dlog
    out = (_mm(x[...], Px[...]) + _mm(agg, P1[...]) + _mm(f1 * agg, P2[...])
           + _mm(f2 * agg, P3[...]) + post_b[...])
    hn = _mm(out, lin_W[...]) + lin_b[...]
    hn_out[...] = hn
    ohT = (lax.broadcasted_iota(jnp.int32, (B, RB), 0) == bc[...]).astype(
        jnp.float32)
    rows = jnp.concatenate([jnp.sum(hn, axis=1, keepdims=True),
                            jnp.sum(hn * hn, axis=1, keepdims=True)], axis=1)
    gstat_out[...] = _mm_exact(ohT, rows).reshape(1, B, 2)


def _layer_agg(x, yd, G, elen, batch_r, batch_c, ee_Wl, ee_bl, We,
               Px, P1, P2, P3, post_bl, lin_Wl, lin_bl):
    blk = lambda r, c: pl.BlockSpec((r, c), lambda i: (i, 0))
    full = lambda r, c: pl.BlockSpec((r, c), lambda i: (0, 0))
    return pl.pallas_call(
        _agg_body,
        grid=(NBLK,),
        in_specs=[blk(RB, HID), blk(RB, HID),
                  pl.BlockSpec((RB * SLOTS, HID), lambda i: (i, 0)),
                  blk(RB, SLOTS), blk(RB, 1),
                  pl.BlockSpec((1, RB), lambda i: (0, i)),
                  full(NBASIS, HID), full(1, HID), full(HID, HID),
                  full(HID, HID),
                  full(4 * HID, HID), full(4 * HID, HID), full(4 * HID, HID),
                  full(1, HID), full(HID, HID), full(1, HID)],
        out_specs=[pl.BlockSpec((RB, HID), lambda i: (i, 0)),
                   pl.BlockSpec((1, B, 2), lambda i: (i, 0, 0))],
        out_shape=[jax.ShapeDtypeStruct((N, HID), jnp.float32),
                   jax.ShapeDtypeStruct((NBLK, B, 2), jnp.float32)],
    )(x, yd, G, elen, batch_r, batch_c, ee_Wl, ee_bl.reshape(1, HID), We,
      Px, P1, P2, P3, post_bl.reshape(1, HID), lin_Wl,
      lin_bl.reshape(1, HID))


def _norm_body(hn, hres, br, gstat, elem, lnw, lnb, h_out):
    tot3 = jnp.sum(gstat[...], axis=0, keepdims=False)
    tot = tot3[:, 0:1]
    tot2 = tot3[:, 1:2]
    inv_elem = 1.0 / elem[...]
    gmean = tot * inv_elem
    gvar = tot2 * inv_elem - gmean * gmean
    rstd = 1.0 / jnp.sqrt(gvar + 1e-5)
    stats = jnp.concatenate([gmean, rstd], axis=1)
    oh = (br[...] == lax.broadcasted_iota(jnp.int32, (RB, B), 1)).astype(
        jnp.float32)
    nst = _mm_exact(oh, stats)
    nmean = nst[:, 0:1]
    nrstd = nst[:, 1:2]
    hnorm = (hn[...] - nmean) * nrstd * lnw[...] + lnb[...]
    a = hnorm + hres[...]
    h_out[...] = a * jax.nn.sigmoid(a)


def _layer_norm(hn, hres, batch_r, gstat, elem, lnw, lnb):
    blk = lambda r, c: pl.BlockSpec((r, c), lambda i: (i, 0))
    full = lambda r, c: pl.BlockSpec((r, c), lambda i: (0, 0))
    return pl.pallas_call(
        _norm_body,
        grid=(NBLK,),
        in_specs=[blk(RB, HID), blk(RB, HID), blk(RB, 1),
                  pl.BlockSpec((NBLK, B, 2), lambda i: (0, 0, 0)),
                  full(B, 1), full(1, HID), full(1, HID)],
        out_specs=pl.BlockSpec((RB, HID), lambda i: (i, 0)),
        out_shape=jax.ShapeDtypeStruct((N, HID), jnp.float32),
    )(hn, hres, batch_r, gstat, elem, lnw.reshape(1, HID),
      lnb.reshape(1, HID))


def _final_body(h, out_W, out_b, o):
    o[...] = _mm(h[...], out_W[...]) + out_b[...]


def _final(h, out_W, out_b):
    return pl.pallas_call(
        _final_body,
        out_shape=jax.ShapeDtypeStruct((N, 3), jnp.float32),
    )(h, out_W, out_b.reshape(1, 3))


def kernel(coords, batch, t, Wf, te_W, te_b, in_W, in_b, out_W, out_b,
           tm_W, tm_b, ee_W, ee_b, pre_W, pre_b, post_W, post_b,
           lin_W, lin_b, ln_w, ln_b):
    batch = batch.astype(jnp.int32)
    batch_r = batch.reshape(N, 1)
    batch_c = batch.reshape(1, N)

    src, elen = _neighbor_search(coords, batch)
    src_flat = src.reshape(E)

    h0, tpn_all, elem = _prologue(coords, batch, t, Wf, te_W, te_b,
                                  tm_W, tm_b, in_W, in_b)
    h = h0
    for l in range(NL):
        Wd = pre_W[l, 0 * HID:1 * HID]
        Ws = pre_W[l, 1 * HID:2 * HID]
        We = pre_W[l, 2 * HID:3 * HID]
        x, yd, ys = _layer_tables(
            h, batch_r, tpn_all[l * B:(l + 1) * B], Wd, Ws, pre_b[l])
        G = _sc_gather(ys, src_flat)
        Px = post_W[l, 0:HID]
        P1 = post_W[l, HID + 0 * 4 * HID:HID + 1 * 4 * HID]
        P2 = post_W[l, HID + 1 * 4 * HID:HID + 2 * 4 * HID]
        P3 = post_W[l, HID + 2 * 4 * HID:HID + 3 * 4 * HID]
        hn, gstat = _layer_agg(x, yd, G, elen, batch_r, batch_c,
                               ee_W[l], ee_b[l], We, Px, P1, P2, P3,
                               post_b[l], lin_W[l], lin_b[l])
        h = _layer_norm(hn, h, batch_r, gstat, elem, ln_w[l], ln_b[l])
    return _final(h, out_W, out_b)


# speedup vs baseline: 1.1787x; 1.1787x over previous
import functools

import jax
import jax.numpy as jnp
import numpy as np
from jax import lax
from jax.experimental import pallas as pl
from jax.experimental.pallas import tpu as pltpu
from jax.experimental.pallas import tpu_sc as plsc

N = 4096
B = 16
HID = 128
TED = 128
NL = 5
RADIUS = 1.5
MAXNB = 30
SLOTS = 32
NBASIS = 16
AVG_DEG_LOG = float(np.log(31.0))
RB = 256
NBLK = N // RB
E = N * SLOTS

_vals = np.linspace(0.0, RADIUS, NBASIS + 2)
_STEP = float(_vals[1] - _vals[0])
_CENTERS = np.asarray(_vals[1:-1], dtype=np.float32)


def _mm(a, b):
    return lax.dot(a.astype(jnp.bfloat16), b.astype(jnp.bfloat16),
                   preferred_element_type=jnp.float32)


def _mm_exact(a, b):
    return lax.dot(a, b, precision=lax.Precision.HIGHEST)


def _nbr_body(cxr, cyr, czr, br, cxc, cyc, czc, bc, src_out, elen_out, d_s):
    i = pl.program_id(0)
    dx = cxr[...] - cxc[...]
    dy = cyr[...] - cyc[...]
    dz = czr[...] - czc[...]
    d = jnp.sqrt((dx * dx + dy * dy) + dz * dz)
    rowid = i * RB + lax.broadcasted_iota(jnp.int32, (RB, N), 0)
    colid = lax.broadcasted_iota(jnp.int32, (RB, N), 1)
    invalid = (br[...] != bc[...]) | (rowid == colid)
    d_s[...] = jnp.where(invalid, jnp.inf, d)

    slot = lax.broadcasted_iota(jnp.int32, (RB, SLOTS), 1)

    def body(k, carry):
        vals, idxs = carry
        dcur = d_s[...]
        rowmin = jnp.min(dcur, axis=1, keepdims=True)
        amin = jnp.min(jnp.where(dcur == rowmin, colid, N), axis=1,
                       keepdims=True)
        d_s[...] = jnp.where(colid == amin, jnp.inf, dcur)
        hit = slot == k
        vals = jnp.where(hit, rowmin, vals)
        idxs = jnp.where(hit, amin, idxs)
        return vals, idxs

    vals0 = jnp.full((RB, SLOTS), jnp.inf, jnp.float32)
    idxs0 = jnp.zeros((RB, SLOTS), jnp.int32)
    vals, idxs = lax.fori_loop(0, MAXNB, body, (vals0, idxs0))
    elen_out[...] = vals
    src_out[...] = idxs


def _neighbor_search(coords, batch):
    cxr = coords[:, 0:1]
    cyr = coords[:, 1:2]
    czr = coords[:, 2:3]
    cxc = coords[:, 0].reshape(1, N)
    cyc = coords[:, 1].reshape(1, N)
    czc = coords[:, 2].reshape(1, N)
    br = batch.reshape(N, 1)
    bc = batch.reshape(1, N)
    row_spec = pl.BlockSpec((RB, 1), lambda i: (i, 0))
    col_spec = pl.BlockSpec((1, N), lambda i: (0, 0))
    return pl.pallas_call(
        _nbr_body,
        grid=(NBLK,),
        in_specs=[row_spec, row_spec, row_spec, row_spec,
                  col_spec, col_spec, col_spec, col_spec],
        out_specs=[pl.BlockSpec((RB, SLOTS), lambda i: (i, 0)),
                   pl.BlockSpec((RB, SLOTS), lambda i: (i, 0))],
        out_shape=[jax.ShapeDtypeStruct((N, SLOTS), jnp.int32),
                   jax.ShapeDtypeStruct((N, SLOTS), jnp.float32)],
        scratch_shapes=[pltpu.VMEM((RB, N), jnp.float32)],
    )(cxr, cyr, czr, br, cxc, cyc, czc, bc)


_NW = 32
_BPW = E // _NW
_CH = 256
_NCHUNK = _BPW // _CH


def _sc_gather(table, idx):
    mesh = plsc.VectorSubcoreMesh(core_axis_name="c", subcore_axis_name="s")

    @functools.partial(
        pl.kernel,
        out_type=jax.ShapeDtypeStruct((E, HID), jnp.float32),
        mesh=mesh,
        scratch_types=[
            pltpu.VMEM((_CH,), jnp.int32),
            pltpu.VMEM((_CH,), jnp.int32),
            pltpu.VMEM((_CH, HID), jnp.float32),
            pltpu.VMEM((_CH, HID), jnp.float32),
            pltpu.SemaphoreType.DMA,
            pltpu.SemaphoreType.DMA,
            pltpu.SemaphoreType.DMA,
            pltpu.SemaphoreType.DMA,
        ],
    )
    def gather_k(idx_hbm, table_hbm, out_hbm, iv0, iv1, rv0, rv1,
                 gs0, gs1, ws0, ws1):
        wid = lax.axis_index("s") * 2 + lax.axis_index("c")
        base = wid * _BPW
        ivs = (iv0, iv1)
        rvs = (rv0, rv1)
        gss = (gs0, gs1)
        wss = (ws0, ws1)

        def load_idx(c, b):
            pltpu.sync_copy(idx_hbm.at[pl.ds(base + c * _CH, _CH)], ivs[b])

        def start_gather(b):
            return pltpu.async_copy(table_hbm.at[ivs[b]], rvs[b], gss[b])

        def start_write(c, b):
            return pltpu.async_copy(
                rvs[b], out_hbm.at[pl.ds(base + c * _CH, _CH)], wss[b])

        load_idx(0, 0)
        g = start_gather(0)
        w = [None, None]
        for c in range(_NCHUNK):
            cur = c & 1
            nxt = 1 - cur
            if c + 1 < _NCHUNK:
                load_idx(c + 1, nxt)
            g.wait()
            if c + 1 < _NCHUNK:
                if w[nxt] is not None:
                    w[nxt].wait()
                    w[nxt] = None
                g = start_gather(nxt)
            w[cur] = start_write(c, cur)
        for b in range(2):
            if w[b] is not None:
                w[b].wait()

    return gather_k(idx, table)


def _prologue_body(coords, t, Wf, te_W, te_b, tm_W, tm_b, in_W, in_b, bc,
                   h0, tpn_all, elem):
    xp = (2.0 * jnp.pi) * t[...] * Wf[...]
    tf = jnp.concatenate([jnp.sin(xp), jnp.cos(xp)], axis=-1)
    tf = _mm(tf, te_W[...]) + te_b[...]
    tf = tf * jax.nn.sigmoid(tf)
    for l in range(NL):
        tpn_all[l * B:(l + 1) * B, :] = _mm(tf, tm_W[l]) + tm_b[l]
    h0[...] = _mm(coords[...], in_W[...]) + in_b[...]
    gid = lax.broadcasted_iota(jnp.int32, (B, N), 0)
    cntg = jnp.sum((gid == bc[...]).astype(jnp.float32), axis=1,
                   keepdims=True)
    elem[...] = jnp.maximum(cntg * HID, 1.0)


def _prologue(coords, batch, t, Wf, te_W, te_b, tm_W, tm_b, in_W, in_b):
    return pl.pallas_call(
        _prologue_body,
        out_shape=[jax.ShapeDtypeStruct((N, HID), jnp.float32),
                   jax.ShapeDtypeStruct((NL * B, HID), jnp.float32),
                   jax.ShapeDtypeStruct((B, 1), jnp.float32)],
    )(coords, t.reshape(B, 1), Wf.reshape(1, TED // 2), te_W,
      te_b.reshape(1, TED), tm_W, tm_b, in_W, in_b.reshape(1, HID),
      batch.reshape(1, N))


def _tables_body(h, br, tpn_g, Wd, Ws, pre_b,
                 x_out, yd_out, ys_out):
    oh = (br[...] == lax.broadcasted_iota(jnp.int32, (N, B), 1)).astype(
        jnp.float32)
    x = h[...] + _mm_exact(oh, tpn_g[...])
    x_out[...] = x
    yd_out[...] = _mm(x, Wd[...]) + pre_b[...]
    ys_out[...] = _mm(x, Ws[...])


def _layer_tables(h, batch_r, tpn_g, Wd, Ws, pre_bl):
    return pl.pallas_call(
        _tables_body,
        out_shape=[jax.ShapeDtypeStruct((N, HID), jnp.float32),
                   jax.ShapeDtypeStruct((N, HID), jnp.float32),
                   jax.ShapeDtypeStruct((N, HID), jnp.float32)],
    )(h, batch_r, tpn_g, Wd, Ws, pre_bl.reshape(1, HID))


def _agg_body(x, yd, G, elen, br, bc, ee_W, ee_b, We, Px, P1, P2, P3,
              post_b, lin_W, lin_b, hn_out, gstat_out):
    centers = (lax.broadcasted_iota(jnp.int32, (1, NBASIS), 1).astype(
        jnp.float32) + 1.0) * _STEP
    Gb = G[...]
    ydv = yd[...]
    elen_v = elen[...]
    basis_list = []
    for j in range(MAXNB):
        elen_j = elen_v[:, j:j + 1]
        diff = (elen_j - centers) * (1.0 / _STEP)
        basis_list.append(jnp.exp(-(diff * diff)) * (1.0 / 1.12))
    basis_all = jnp.concatenate(basis_list, axis=0)
    e_all = _mm(basis_all, ee_W[...]) + ee_b[...]
    E3 = _mm(e_all, We[...]).reshape(MAXNB, RB, HID)
    s1 = jnp.zeros((RB, HID), jnp.float32)
    s2 = jnp.zeros((RB, HID), jnp.float32)
    mx = jnp.full((RB, HID), -jnp.inf, jnp.float32)
    mn = jnp.full((RB, HID), jnp.inf, jnp.float32)
    cnt = jnp.zeros((RB, 1), jnp.float32)
    for j in range(MAXNB):
        elen_j = elen_v[:, j:j + 1]
        m_j = ydv + Gb[j] + E3[j]
        ok = elen_j <= RADIUS
        w_j = ok.astype(jnp.float32)
        s1 = s1 + w_j * m_j
        s2 = s2 + w_j * (m_j * m_j)
        mx = jnp.maximum(mx, jnp.where(ok, m_j, -jnp.inf))
        mn = jnp.minimum(mn, jnp.where(ok, m_j, jnp.inf))
        cnt = cnt + w_j
    cntc = jnp.maximum(cnt, 1.0)
    inv = 1.0 / cntc
    mean = s1 * inv
    mean2 = s2 * inv
    std = jnp.sqrt(jax.nn.relu(mean2 - mean * mean) + 1e-5)
    has = cnt > 0.0
    mx = jnp.where(has, mx, 0.0)
    mn = jnp.where(has, mn, 0.0)
    agg = jnp.concatenate([mean, mx, mn, std], axis=-1)
    dlog = jnp.log(cntc + 1.0)
    f1 = dlog * (1.0 / AVG_DEG_LOG)
    f2 = AVG_DEG_LOG / dlog
    out = (_mm(x[...], Px[...]) + _mm(agg, P1[...]) + _mm(f1 * agg, P2[...])
           + _mm(f2 * agg, P3[...]) + post_b[...])
    hn = _mm(out, lin_W[...]) + lin_b[...]
    hn_out[...] = hn
    ohT = (lax.broadcasted_iota(jnp.int32, (B, RB), 0) == bc[...]).astype(
        jnp.float32)
    rows = jnp.concatenate([jnp.sum(hn, axis=1, keepdims=True),
                            jnp.sum(hn * hn, axis=1, keepdims=True)], axis=1)
    gstat_out[...] = _mm_exact(ohT, rows).reshape(1, B, 2)


def _layer_agg(x, yd, G, elen, batch_r, batch_c, ee_Wl, ee_bl, We,
               Px, P1, P2, P3, post_bl, lin_Wl, lin_bl):
    blk = lambda r, c: pl.BlockSpec((r, c), lambda i: (i, 0))
    full = lambda r, c: pl.BlockSpec((r, c), lambda i: (0, 0))
    return pl.pallas_call(
        _agg_body,
        grid=(NBLK,),
        in_specs=[blk(RB, HID), blk(RB, HID),
                  pl.BlockSpec((SLOTS, RB, HID), lambda i: (0, i, 0)),
                  blk(RB, SLOTS), blk(RB, 1),
                  pl.BlockSpec((1, RB), lambda i: (0, i)),
                  full(NBASIS, HID), full(1, HID), full(HID, HID),
                  full(HID, HID),
                  full(4 * HID, HID), full(4 * HID, HID), full(4 * HID, HID),
                  full(1, HID), full(HID, HID), full(1, HID)],
        out_specs=[pl.BlockSpec((RB, HID), lambda i: (i, 0)),
                   pl.BlockSpec((1, B, 2), lambda i: (i, 0, 0))],
        out_shape=[jax.ShapeDtypeStruct((N, HID), jnp.float32),
                   jax.ShapeDtypeStruct((NBLK, B, 2), jnp.float32)],
    )(x, yd, G, elen, batch_r, batch_c, ee_Wl, ee_bl.reshape(1, HID), We,
      Px, P1, P2, P3, post_bl.reshape(1, HID), lin_Wl,
      lin_bl.reshape(1, HID))


def _graph_stats(gstat, elem):
    tot3 = jnp.sum(gstat[...], axis=0, keepdims=False)
    inv_elem = 1.0 / elem[...]
    gmean = tot3[:, 0:1] * inv_elem
    gvar = tot3[:, 1:2] * inv_elem - gmean * gmean
    rstd = 1.0 / jnp.sqrt(gvar + 1e-5)
    return jnp.concatenate([gmean, rstd], axis=1)


def _norm_tables_body(hn, hres, br, gstat, elem, lnw, lnb, tpn_g, Wd, Ws,
                      pre_b, h_out, x_out, yd_out, ys_out):
    stats = _graph_stats(gstat, elem)
    oh = (br[...] == lax.broadcasted_iota(jnp.int32, (N, B), 1)).astype(
        jnp.float32)
    nst = _mm_exact(oh, stats)
    hnorm = (hn[...] - nst[:, 0:1]) * nst[:, 1:2] * lnw[...] + lnb[...]
    a = hnorm + hres[...]
    h = a * jax.nn.sigmoid(a)
    h_out[...] = h
    x = h + _mm_exact(oh, tpn_g[...])
    x_out[...] = x
    yd_out[...] = _mm(x, Wd[...]) + pre_b[...]
    ys_out[...] = _mm(x, Ws[...])


def _layer_norm_tables(hn, hres, batch_r, gstat, elem, lnw, lnb, tpn_g,
                       Wd, Ws, pre_bl):
    return pl.pallas_call(
        _norm_tables_body,
        out_shape=[jax.ShapeDtypeStruct((N, HID), jnp.float32),
                   jax.ShapeDtypeStruct((N, HID), jnp.float32),
                   jax.ShapeDtypeStruct((N, HID), jnp.float32),
                   jax.ShapeDtypeStruct((N, HID), jnp.float32)],
    )(hn, hres, batch_r, gstat, elem, lnw.reshape(1, HID),
      lnb.reshape(1, HID), tpn_g, Wd, Ws, pre_bl.reshape(1, HID))


def _norm_final_body(hn, hres, br, gstat, elem, lnw, lnb, out_W, out_b, o):
    stats = _graph_stats(gstat, elem)
    oh = (br[...] == lax.broadcasted_iota(jnp.int32, (N, B), 1)).astype(
        jnp.float32)
    nst = _mm_exact(oh, stats)
    hnorm = (hn[...] - nst[:, 0:1]) * nst[:, 1:2] * lnw[...] + lnb[...]
    a = hnorm + hres[...]
    h = a * jax.nn.sigmoid(a)
    o[...] = _mm(h, out_W[...]) + out_b[...]


def _layer_norm_final(hn, hres, batch_r, gstat, elem, lnw, lnb, out_W, out_b):
    return pl.pallas_call(
        _norm_final_body,
        out_shape=jax.ShapeDtypeStruct((N, 3), jnp.float32),
    )(hn, hres, batch_r, gstat, elem, lnw.reshape(1, HID),
      lnb.reshape(1, HID), out_W, out_b.reshape(1, 3))


def _norm_body(hn, hres, br, gstat, elem, lnw, lnb, h_out):
    tot3 = jnp.sum(gstat[...], axis=0, keepdims=False)
    tot = tot3[:, 0:1]
    tot2 = tot3[:, 1:2]
    inv_elem = 1.0 / elem[...]
    gmean = tot * inv_elem
    gvar = tot2 * inv_elem - gmean * gmean
    rstd = 1.0 / jnp.sqrt(gvar + 1e-5)
    stats = jnp.concatenate([gmean, rstd], axis=1)
    oh = (br[...] == lax.broadcasted_iota(jnp.int32, (RB, B), 1)).astype(
        jnp.float32)
    nst = _mm_exact(oh, stats)
    nmean = nst[:, 0:1]
    nrstd = nst[:, 1:2]
    hnorm = (hn[...] - nmean) * nrstd * lnw[...] + lnb[...]
    a = hnorm + hres[...]
    h_out[...] = a * jax.nn.sigmoid(a)


def _layer_norm(hn, hres, batch_r, gstat, elem, lnw, lnb):
    blk = lambda r, c: pl.BlockSpec((r, c), lambda i: (i, 0))
    full = lambda r, c: pl.BlockSpec((r, c), lambda i: (0, 0))
    return pl.pallas_call(
        _norm_body,
        grid=(NBLK,),
        in_specs=[blk(RB, HID), blk(RB, HID), blk(RB, 1),
                  pl.BlockSpec((NBLK, B, 2), lambda i: (0, 0, 0)),
                  full(B, 1), full(1, HID), full(1, HID)],
        out_specs=pl.BlockSpec((RB, HID), lambda i: (i, 0)),
        out_shape=jax.ShapeDtypeStruct((N, HID), jnp.float32),
    )(hn, hres, batch_r, gstat, elem, lnw.reshape(1, HID),
      lnb.reshape(1, HID))


def _final_body(h, out_W, out_b, o):
    o[...] = _mm(h[...], out_W[...]) + out_b[...]


def _final(h, out_W, out_b):
    return pl.pallas_call(
        _final_body,
        out_shape=jax.ShapeDtypeStruct((N, 3), jnp.float32),
    )(h, out_W, out_b.reshape(1, 3))


def kernel(coords, batch, t, Wf, te_W, te_b, in_W, in_b, out_W, out_b,
           tm_W, tm_b, ee_W, ee_b, pre_W, pre_b, post_W, post_b,
           lin_W, lin_b, ln_w, ln_b):
    batch = batch.astype(jnp.int32)
    batch_r = batch.reshape(N, 1)
    batch_c = batch.reshape(1, N)

    src, elen = _neighbor_search(coords, batch)
    src_flat = src.T.reshape(E)

    h0, tpn_all, elem = _prologue(coords, batch, t, Wf, te_W, te_b,
                                  tm_W, tm_b, in_W, in_b)
    h = h0
    Wd = [pre_W[l, 0 * HID:1 * HID] for l in range(NL)]
    Ws = [pre_W[l, 1 * HID:2 * HID] for l in range(NL)]
    We = [pre_W[l, 2 * HID:3 * HID] for l in range(NL)]
    x, yd, ys = _layer_tables(h0, batch_r, tpn_all[0:B], Wd[0], Ws[0],
                              pre_b[0])
    for l in range(NL):
        G = _sc_gather(ys, src_flat).reshape(SLOTS, N, HID)
        Px = post_W[l, 0:HID]
        P1 = post_W[l, HID + 0 * 4 * HID:HID + 1 * 4 * HID]
        P2 = post_W[l, HID + 1 * 4 * HID:HID + 2 * 4 * HID]
        P3 = post_W[l, HID + 2 * 4 * HID:HID + 3 * 4 * HID]
        hn, gstat = _layer_agg(x, yd, G, elen, batch_r, batch_c,
                               ee_W[l], ee_b[l], We[l], Px, P1, P2, P3,
                               post_b[l], lin_W[l], lin_b[l])
        if l + 1 < NL:
            h, x, yd, ys = _layer_norm_tables(
                hn, h, batch_r, gstat, elem, ln_w[l], ln_b[l],
                tpn_all[(l + 1) * B:(l + 2) * B], Wd[l + 1], Ws[l + 1],
                pre_b[l + 1])
        else:
            return _layer_norm_final(hn, h, batch_r, gstat, elem,
                                     ln_w[l], ln_b[l], out_W, out_b)


# speedup vs baseline: 1.1825x; 1.0032x over previous
import functools

import jax
import jax.numpy as jnp
import numpy as np
from jax import lax
from jax.experimental import pallas as pl
from jax.experimental.pallas import tpu as pltpu
from jax.experimental.pallas import tpu_sc as plsc

N = 4096
B = 16
HID = 128
TED = 128
NL = 5
RADIUS = 1.5
MAXNB = 30
SLOTS = 32
NBASIS = 16
AVG_DEG_LOG = float(np.log(31.0))
RB = 256
NBLK = N // RB
E = N * SLOTS

_vals = np.linspace(0.0, RADIUS, NBASIS + 2)
_STEP = float(_vals[1] - _vals[0])
_CENTERS = np.asarray(_vals[1:-1], dtype=np.float32)


def _mm(a, b):
    return lax.dot(a.astype(jnp.bfloat16), b.astype(jnp.bfloat16),
                   preferred_element_type=jnp.float32)


def _mm_exact(a, b):
    return lax.dot(a, b, precision=lax.Precision.HIGHEST)


def _nbr_body(cxr, cyr, czr, br, cxc, cyc, czc, bc, src_out, elen_out, d_s):
    i = pl.program_id(0)
    dx = cxr[...] - cxc[...]
    dy = cyr[...] - cyc[...]
    dz = czr[...] - czc[...]
    d = jnp.sqrt((dx * dx + dy * dy) + dz * dz)
    rowid = i * RB + lax.broadcasted_iota(jnp.int32, (RB, N), 0)
    colid = lax.broadcasted_iota(jnp.int32, (RB, N), 1)
    invalid = (br[...] != bc[...]) | (rowid == colid)
    d_s[...] = jnp.where(invalid, jnp.inf, d)

    slot = lax.broadcasted_iota(jnp.int32, (RB, SLOTS), 1)

    def body(k, carry):
        vals, idxs = carry
        dcur = d_s[...]
        rowmin = jnp.min(dcur, axis=1, keepdims=True)
        amin = jnp.min(jnp.where(dcur == rowmin, colid, N), axis=1,
                       keepdims=True)
        d_s[...] = jnp.where(colid == amin, jnp.inf, dcur)
        hit = slot == k
        vals = jnp.where(hit, rowmin, vals)
        idxs = jnp.where(hit, amin, idxs)
        return vals, idxs

    vals0 = jnp.full((RB, SLOTS), jnp.inf, jnp.float32)
    idxs0 = jnp.zeros((RB, SLOTS), jnp.int32)
    vals, idxs = lax.fori_loop(0, MAXNB, body, (vals0, idxs0))
    elen_out[...] = vals
    src_out[...] = idxs


def _neighbor_search(coords, batch):
    cxr = coords[:, 0:1]
    cyr = coords[:, 1:2]
    czr = coords[:, 2:3]
    cxc = coords[:, 0].reshape(1, N)
    cyc = coords[:, 1].reshape(1, N)
    czc = coords[:, 2].reshape(1, N)
    br = batch.reshape(N, 1)
    bc = batch.reshape(1, N)
    row_spec = pl.BlockSpec((RB, 1), lambda i: (i, 0))
    col_spec = pl.BlockSpec((1, N), lambda i: (0, 0))
    return pl.pallas_call(
        _nbr_body,
        grid=(NBLK,),
        in_specs=[row_spec, row_spec, row_spec, row_spec,
                  col_spec, col_spec, col_spec, col_spec],
        out_specs=[pl.BlockSpec((RB, SLOTS), lambda i: (i, 0)),
                   pl.BlockSpec((RB, SLOTS), lambda i: (i, 0))],
        out_shape=[jax.ShapeDtypeStruct((N, SLOTS), jnp.int32),
                   jax.ShapeDtypeStruct((N, SLOTS), jnp.float32)],
        scratch_shapes=[pltpu.VMEM((RB, N), jnp.float32)],
    )(cxr, cyr, czr, br, cxc, cyc, czc, bc)


_NW = 32
_BPW = E // _NW
_CH = 128
_NCHUNK = _BPW // _CH
_NBUF = 4
_DEPTH = 3


def _sc_gather(table, idx):
    mesh = plsc.VectorSubcoreMesh(core_axis_name="c", subcore_axis_name="s")

    @functools.partial(
        pl.kernel,
        out_type=jax.ShapeDtypeStruct((E, HID), jnp.float32),
        mesh=mesh,
        scratch_types=(
            [pltpu.VMEM((_CH,), jnp.int32) for _ in range(_NBUF)]
            + [pltpu.VMEM((_CH, HID), jnp.float32) for _ in range(_NBUF)]
            + [pltpu.SemaphoreType.DMA for _ in range(2 * _NBUF)]
        ),
    )
    def gather_k(idx_hbm, table_hbm, out_hbm, *bufs):
        ivs = bufs[0:_NBUF]
        rvs = bufs[_NBUF:2 * _NBUF]
        gss = bufs[2 * _NBUF:3 * _NBUF]
        wss = bufs[3 * _NBUF:4 * _NBUF]
        wid = lax.axis_index("s") * 2 + lax.axis_index("c")
        base = wid * _BPW

        def load_idx(c, b):
            pltpu.sync_copy(idx_hbm.at[pl.ds(base + c * _CH, _CH)], ivs[b])

        def start_gather(b):
            return pltpu.async_copy(table_hbm.at[ivs[b]], rvs[b], gss[b])

        def start_write(c, b):
            return pltpu.async_copy(
                rvs[b], out_hbm.at[pl.ds(base + c * _CH, _CH)], wss[b])

        g = [None] * _NBUF
        w = [None] * _NBUF
        for c in range(_DEPTH):
            load_idx(c, c % _NBUF)
            g[c % _NBUF] = start_gather(c % _NBUF)
        for c in range(_NCHUNK):
            b = c % _NBUF
            g[b].wait()
            g[b] = None
            w[b] = start_write(c, b)
            n = c + _DEPTH
            if n < _NCHUNK:
                nb = n % _NBUF
                if w[nb] is not None:
                    w[nb].wait()
                    w[nb] = None
                load_idx(n, nb)
                g[nb] = start_gather(nb)
        for b in range(_NBUF):
            if w[b] is not None:
                w[b].wait()

    return gather_k(idx, table)


def _prologue_body(coords, t, Wf, te_W, te_b, tm_W, tm_b, in_W, in_b, bc,
                   h0, tpn_all, elem):
    xp = (2.0 * jnp.pi) * t[...] * Wf[...]
    tf = jnp.concatenate([jnp.sin(xp), jnp.cos(xp)], axis=-1)
    tf = _mm(tf, te_W[...]) + te_b[...]
    tf = tf * jax.nn.sigmoid(tf)
    for l in range(NL):
        tpn_all[l * B:(l + 1) * B, :] = _mm(tf, tm_W[l]) + tm_b[l]
    h0[...] = _mm(coords[...], in_W[...]) + in_b[...]
    gid = lax.broadcasted_iota(jnp.int32, (B, N), 0)
    cntg = jnp.sum((gid == bc[...]).astype(jnp.float32), axis=1,
                   keepdims=True)
    elem[...] = jnp.maximum(cntg * HID, 1.0)


def _prologue(coords, batch, t, Wf, te_W, te_b, tm_W, tm_b, in_W, in_b):
    return pl.pallas_call(
        _prologue_body,
        out_shape=[jax.ShapeDtypeStruct((N, HID), jnp.float32),
                   jax.ShapeDtypeStruct((NL * B, HID), jnp.float32),
                   jax.ShapeDtypeStruct((B, 1), jnp.float32)],
    )(coords, t.reshape(B, 1), Wf.reshape(1, TED // 2), te_W,
      te_b.reshape(1, TED), tm_W, tm_b, in_W, in_b.reshape(1, HID),
      batch.reshape(1, N))


def _tables_body(h, br, tpn_g, Wd, Ws, pre_b,
                 x_out, yd_out, ys_out):
    oh = (br[...] == lax.broadcasted_iota(jnp.int32, (N, B), 1)).astype(
        jnp.float32)
    x = h[...] + _mm_exact(oh, tpn_g[...])
    x_out[...] = x
    yd_out[...] = _mm(x, Wd[...]) + pre_b[...]
    ys_out[...] = _mm(x, Ws[...])


def _layer_tables(h, batch_r, tpn_g, Wd, Ws, pre_bl):
    return pl.pallas_call(
        _tables_body,
        out_shape=[jax.ShapeDtypeStruct((N, HID), jnp.float32),
                   jax.ShapeDtypeStruct((N, HID), jnp.float32),
                   jax.ShapeDtypeStruct((N, HID), jnp.float32)],
    )(h, batch_r, tpn_g, Wd, Ws, pre_bl.reshape(1, HID))


def _agg_body(x, yd, G, elen, br, bc, ee_W, ee_b, We, Px, P1, P2, P3,
              post_b, lin_W, lin_b, hn_out, gstat_out):
    centers = (lax.broadcasted_iota(jnp.int32, (1, NBASIS), 1).astype(
        jnp.float32) + 1.0) * _STEP
    Gb = G[...]
    ydv = yd[...]
    elen_v = elen[...]
    basis_list = []
    for j in range(MAXNB):
        elen_j = elen_v[:, j:j + 1]
        diff = (elen_j - centers) * (1.0 / _STEP)
        basis_list.append(jnp.exp(-(diff * diff)) * (1.0 / 1.12))
    basis_all = jnp.concatenate(basis_list, axis=0)
    e_all = _mm(basis_all, ee_W[...]) + ee_b[...]
    E3 = _mm(e_all, We[...]).reshape(MAXNB, RB, HID)
    s1 = jnp.zeros((RB, HID), jnp.float32)
    s2 = jnp.zeros((RB, HID), jnp.float32)
    mx = jnp.full((RB, HID), -jnp.inf, jnp.float32)
    mn = jnp.full((RB, HID), jnp.inf, jnp.float32)
    cnt = jnp.zeros((RB, 1), jnp.float32)
    for j in range(MAXNB):
        elen_j = elen_v[:, j:j + 1]
        m_j = ydv + Gb[j] + E3[j]
        ok = elen_j <= RADIUS
        w_j = ok.astype(jnp.float32)
        s1 = s1 + w_j * m_j
        s2 = s2 + w_j * (m_j * m_j)
        mx = jnp.maximum(mx, jnp.where(ok, m_j, -jnp.inf))
        mn = jnp.minimum(mn, jnp.where(ok, m_j, jnp.inf))
        cnt = cnt + w_j
    cntc = jnp.maximum(cnt, 1.0)
    inv = 1.0 / cntc
    mean = s1 * inv
    mean2 = s2 * inv
    std = jnp.sqrt(jax.nn.relu(mean2 - mean * mean) + 1e-5)
    has = cnt > 0.0
    mx = jnp.where(has, mx, 0.0)
    mn = jnp.where(has, mn, 0.0)
    agg = jnp.concatenate([mean, mx, mn, std], axis=-1)
    dlog = jnp.log(cntc + 1.0)
    f1 = dlog * (1.0 / AVG_DEG_LOG)
    f2 = AVG_DEG_LOG / dlog
    out = (_mm(x[...], Px[...]) + _mm(agg, P1[...]) + _mm(f1 * agg, P2[...])
           + _mm(f2 * agg, P3[...]) + post_b[...])
    hn = _mm(out, lin_W[...]) + lin_b[...]
    hn_out[...] = hn
    ohT = (lax.broadcasted_iota(jnp.int32, (B, RB), 0) == bc[...]).astype(
        jnp.float32)
    rows = jnp.concatenate([jnp.sum(hn, axis=1, keepdims=True),
                            jnp.sum(hn * hn, axis=1, keepdims=True)], axis=1)
    gstat_out[...] = _mm_exact(ohT, rows).reshape(1, B, 2)


def _layer_agg(x, yd, G, elen, batch_r, batch_c, ee_Wl, ee_bl, We,
               Px, P1, P2, P3, post_bl, lin_Wl, lin_bl):
    blk = lambda r, c: pl.BlockSpec((r, c), lambda i: (i, 0))
    full = lambda r, c: pl.BlockSpec((r, c), lambda i: (0, 0))
    return pl.pallas_call(
        _agg_body,
        grid=(NBLK,),
        in_specs=[blk(RB, HID), blk(RB, HID),
                  pl.BlockSpec((SLOTS, RB, HID), lambda i: (0, i, 0)),
                  blk(RB, SLOTS), blk(RB, 1),
                  pl.BlockSpec((1, RB), lambda i: (0, i)),
                  full(NBASIS, HID), full(1, HID), full(HID, HID),
                  full(HID, HID),
                  full(4 * HID, HID), full(4 * HID, HID), full(4 * HID, HID),
                  full(1, HID), full(HID, HID), full(1, HID)],
        out_specs=[pl.BlockSpec((RB, HID), lambda i: (i, 0)),
                   pl.BlockSpec((1, B, 2), lambda i: (i, 0, 0))],
        out_shape=[jax.ShapeDtypeStruct((N, HID), jnp.float32),
                   jax.ShapeDtypeStruct((NBLK, B, 2), jnp.float32)],
    )(x, yd, G, elen, batch_r, batch_c, ee_Wl, ee_bl.reshape(1, HID), We,
      Px, P1, P2, P3, post_bl.reshape(1, HID), lin_Wl,
      lin_bl.reshape(1, HID))


def _graph_stats(gstat, elem):
    tot3 = jnp.sum(gstat[...], axis=0, keepdims=False)
    inv_elem = 1.0 / elem[...]
    gmean = tot3[:, 0:1] * inv_elem
    gvar = tot3[:, 1:2] * inv_elem - gmean * gmean
    rstd = 1.0 / jnp.sqrt(gvar + 1e-5)
    return jnp.concatenate([gmean, rstd], axis=1)


def _norm_tables_body(hn, hres, br, gstat, elem, lnw, lnb, tpn_g, Wd, Ws,
                      pre_b, h_out, x_out, yd_out, ys_out):
    stats = _graph_stats(gstat, elem)
    oh = (br[...] == lax.broadcasted_iota(jnp.int32, (N, B), 1)).astype(
        jnp.float32)
    nst = _mm_exact(oh, stats)
    hnorm = (hn[...] - nst[:, 0:1]) * nst[:, 1:2] * lnw[...] + lnb[...]
    a = hnorm + hres[...]
    h = a * jax.nn.sigmoid(a)
    h_out[...] = h
    x = h + _mm_exact(oh, tpn_g[...])
    x_out[...] = x
    yd_out[...] = _mm(x, Wd[...]) + pre_b[...]
    ys_out[...] = _mm(x, Ws[...])


def _layer_norm_tables(hn, hres, batch_r, gstat, elem, lnw, lnb, tpn_g,
                       Wd, Ws, pre_bl):
    return pl.pallas_call(
        _norm_tables_body,
        out_shape=[jax.ShapeDtypeStruct((N, HID), jnp.float32),
                   jax.ShapeDtypeStruct((N, HID), jnp.float32),
                   jax.ShapeDtypeStruct((N, HID), jnp.float32),
                   jax.ShapeDtypeStruct((N, HID), jnp.float32)],
    )(hn, hres, batch_r, gstat, elem, lnw.reshape(1, HID),
      lnb.reshape(1, HID), tpn_g, Wd, Ws, pre_bl.reshape(1, HID))


def _norm_final_body(hn, hres, br, gstat, elem, lnw, lnb, out_W, out_b, o):
    stats = _graph_stats(gstat, elem)
    oh = (br[...] == lax.broadcasted_iota(jnp.int32, (N, B), 1)).astype(
        jnp.float32)
    nst = _mm_exact(oh, stats)
    hnorm = (hn[...] - nst[:, 0:1]) * nst[:, 1:2] * lnw[...] + lnb[...]
    a = hnorm + hres[...]
    h = a * jax.nn.sigmoid(a)
    o[...] = _mm(h, out_W[...]) + out_b[...]


def _layer_norm_final(hn, hres, batch_r, gstat, elem, lnw, lnb, out_W, out_b):
    return pl.pallas_call(
        _norm_final_body,
        out_shape=jax.ShapeDtypeStruct((N, 3), jnp.float32),
    )(hn, hres, batch_r, gstat, elem, lnw.reshape(1, HID),
      lnb.reshape(1, HID), out_W, out_b.reshape(1, 3))


def _norm_body(hn, hres, br, gstat, elem, lnw, lnb, h_out):
    tot3 = jnp.sum(gstat[...], axis=0, keepdims=False)
    tot = tot3[:, 0:1]
    tot2 = tot3[:, 1:2]
    inv_elem = 1.0 / elem[...]
    gmean = tot * inv_elem
    gvar = tot2 * inv_elem - gmean * gmean
    rstd = 1.0 / jnp.sqrt(gvar + 1e-5)
    stats = jnp.concatenate([gmean, rstd], axis=1)
    oh = (br[...] == lax.broadcasted_iota(jnp.int32, (RB, B), 1)).astype(
        jnp.float32)
    nst = _mm_exact(oh, stats)
    nmean = nst[:, 0:1]
    nrstd = nst[:, 1:2]
    hnorm = (hn[...] - nmean) * nrstd * lnw[...] + lnb[...]
    a = hnorm + hres[...]
    h_out[...] = a * jax.nn.sigmoid(a)


def _layer_norm(hn, hres, batch_r, gstat, elem, lnw, lnb):
    blk = lambda r, c: pl.BlockSpec((r, c), lambda i: (i, 0))
    full = lambda r, c: pl.BlockSpec((r, c), lambda i: (0, 0))
    return pl.pallas_call(
        _norm_body,
        grid=(NBLK,),
        in_specs=[blk(RB, HID), blk(RB, HID), blk(RB, 1),
                  pl.BlockSpec((NBLK, B, 2), lambda i: (0, 0, 0)),
                  full(B, 1), full(1, HID), full(1, HID)],
        out_specs=pl.BlockSpec((RB, HID), lambda i: (i, 0)),
        out_shape=jax.ShapeDtypeStruct((N, HID), jnp.float32),
    )(hn, hres, batch_r, gstat, elem, lnw.reshape(1, HID),
      lnb.reshape(1, HID))


def _final_body(h, out_W, out_b, o):
    o[...] = _mm(h[...], out_W[...]) + out_b[...]


def _final(h, out_W, out_b):
    return pl.pallas_call(
        _final_body,
        out_shape=jax.ShapeDtypeStruct((N, 3), jnp.float32),
    )(h, out_W, out_b.reshape(1, 3))


def kernel(coords, batch, t, Wf, te_W, te_b, in_W, in_b, out_W, out_b,
           tm_W, tm_b, ee_W, ee_b, pre_W, pre_b, post_W, post_b,
           lin_W, lin_b, ln_w, ln_b):
    batch = batch.astype(jnp.int32)
    batch_r = batch.reshape(N, 1)
    batch_c = batch.reshape(1, N)

    src, elen = _neighbor_search(coords, batch)
    src_flat = src.T.reshape(E)

    h0, tpn_all, elem = _prologue(coords, batch, t, Wf, te_W, te_b,
                                  tm_W, tm_b, in_W, in_b)
    h = h0
    Wd = [pre_W[l, 0 * HID:1 * HID] for l in range(NL)]
    Ws = [pre_W[l, 1 * HID:2 * HID] for l in range(NL)]
    We = [pre_W[l, 2 * HID:3 * HID] for l in range(NL)]
    x, yd, ys = _layer_tables(h0, batch_r, tpn_all[0:B], Wd[0], Ws[0],
                              pre_b[0])
    for l in range(NL):
        G = _sc_gather(ys, src_flat).reshape(SLOTS, N, HID)
        Px = post_W[l, 0:HID]
        P1 = post_W[l, HID + 0 * 4 * HID:HID + 1 * 4 * HID]
        P2 = post_W[l, HID + 1 * 4 * HID:HID + 2 * 4 * HID]
        P3 = post_W[l, HID + 2 * 4 * HID:HID + 3 * 4 * HID]
        hn, gstat = _layer_agg(x, yd, G, elen, batch_r, batch_c,
                               ee_W[l], ee_b[l], We[l], Px, P1, P2, P3,
                               post_b[l], lin_W[l], lin_b[l])
        if l + 1 < NL:
            h, x, yd, ys = _layer_norm_tables(
                hn, h, batch_r, gstat, elem, ln_w[l], ln_b[l],
                tpn_all[(l + 1) * B:(l + 2) * B], Wd[l + 1], Ws[l + 1],
                pre_b[l + 1])
        else:
            return _layer_norm_final(hn, h, batch_r, gstat, elem,
                                     ln_w[l], ln_b[l], out_W, out_b)


# speedup vs baseline: 1.2237x; 1.0349x over previous
import functools

import jax
import jax.numpy as jnp
import numpy as np
from jax import lax
from jax.experimental import pallas as pl
from jax.experimental.pallas import tpu as pltpu
from jax.experimental.pallas import tpu_sc as plsc

N = 4096
B = 16
HID = 128
TED = 128
NL = 5
RADIUS = 1.5
MAXNB = 30
SLOTS = 32
NBASIS = 16
AVG_DEG_LOG = float(np.log(31.0))
RB = 256
NBLK = N // RB
E = N * SLOTS

_vals = np.linspace(0.0, RADIUS, NBASIS + 2)
_STEP = float(_vals[1] - _vals[0])
_CENTERS = np.asarray(_vals[1:-1], dtype=np.float32)


def _mm(a, b):
    return lax.dot(a.astype(jnp.bfloat16), b.astype(jnp.bfloat16),
                   preferred_element_type=jnp.float32)


def _mm_exact(a, b):
    return lax.dot(a, b, precision=lax.Precision.HIGHEST)


def _nbr_body(cxr, cyr, czr, br, cxc, cyc, czc, bc, src_out, elen_out, d_s):
    i = pl.program_id(0)
    dx = cxr[...] - cxc[...]
    dy = cyr[...] - cyc[...]
    dz = czr[...] - czc[...]
    d = jnp.sqrt((dx * dx + dy * dy) + dz * dz)
    rowid = i * RB + lax.broadcasted_iota(jnp.int32, (RB, N), 0)
    colid = lax.broadcasted_iota(jnp.int32, (RB, N), 1)
    invalid = (br[...] != bc[...]) | (rowid == colid)
    d_s[...] = jnp.where(invalid, jnp.inf, d)

    slot = lax.broadcasted_iota(jnp.int32, (RB, SLOTS), 1)

    def body(k, carry):
        vals, idxs = carry
        dcur = d_s[...]
        rowmin = jnp.min(dcur, axis=1, keepdims=True)
        amin = jnp.min(jnp.where(dcur == rowmin, colid, N), axis=1,
                       keepdims=True)
        d_s[...] = jnp.where(colid == amin, jnp.inf, dcur)
        hit = slot == k
        vals = jnp.where(hit, rowmin, vals)
        idxs = jnp.where(hit, amin, idxs)
        return vals, idxs

    vals0 = jnp.full((RB, SLOTS), jnp.inf, jnp.float32)
    idxs0 = jnp.zeros((RB, SLOTS), jnp.int32)
    vals, idxs = lax.fori_loop(0, MAXNB, body, (vals0, idxs0))
    elen_out[...] = vals
    src_out[...] = idxs


def _neighbor_search(coords, batch):
    cxr = coords[:, 0:1]
    cyr = coords[:, 1:2]
    czr = coords[:, 2:3]
    cxc = coords[:, 0].reshape(1, N)
    cyc = coords[:, 1].reshape(1, N)
    czc = coords[:, 2].reshape(1, N)
    br = batch.reshape(N, 1)
    bc = batch.reshape(1, N)
    row_spec = pl.BlockSpec((RB, 1), lambda i: (i, 0))
    col_spec = pl.BlockSpec((1, N), lambda i: (0, 0))
    return pl.pallas_call(
        _nbr_body,
        grid=(NBLK,),
        in_specs=[row_spec, row_spec, row_spec, row_spec,
                  col_spec, col_spec, col_spec, col_spec],
        out_specs=[pl.BlockSpec((RB, SLOTS), lambda i: (i, 0)),
                   pl.BlockSpec((RB, SLOTS), lambda i: (i, 0))],
        out_shape=[jax.ShapeDtypeStruct((N, SLOTS), jnp.int32),
                   jax.ShapeDtypeStruct((N, SLOTS), jnp.float32)],
        scratch_shapes=[pltpu.VMEM((RB, N), jnp.float32)],
    )(cxr, cyr, czr, br, cxc, cyc, czc, bc)


_NW = 32
_BPW = E // _NW
_CH = 128
_NCHUNK = _BPW // _CH
_NBUF = 4
_DEPTH = 3


def _sc_gather(table, idx):
    mesh = plsc.VectorSubcoreMesh(core_axis_name="c", subcore_axis_name="s")
    n_rows = idx.shape[0]
    bpw = n_rows // _NW
    nchunk = bpw // _CH

    @functools.partial(
        pl.kernel,
        out_type=jax.ShapeDtypeStruct((n_rows, HID), jnp.float32),
        mesh=mesh,
        scratch_types=(
            [pltpu.VMEM((_CH,), jnp.int32) for _ in range(_NBUF)]
            + [pltpu.VMEM((_CH, HID), jnp.float32) for _ in range(_NBUF)]
            + [pltpu.SemaphoreType.DMA for _ in range(2 * _NBUF)]
        ),
    )
    def gather_k(idx_hbm, table_hbm, out_hbm, *bufs):
        ivs = bufs[0:_NBUF]
        rvs = bufs[_NBUF:2 * _NBUF]
        gss = bufs[2 * _NBUF:3 * _NBUF]
        wss = bufs[3 * _NBUF:4 * _NBUF]
        wid = lax.axis_index("s") * 2 + lax.axis_index("c")
        base = wid * bpw

        def load_idx(c, b):
            pltpu.sync_copy(idx_hbm.at[pl.ds(base + c * _CH, _CH)], ivs[b])

        def start_gather(b):
            return pltpu.async_copy(table_hbm.at[ivs[b]], rvs[b], gss[b])

        def start_write(c, b):
            return pltpu.async_copy(
                rvs[b], out_hbm.at[pl.ds(base + c * _CH, _CH)], wss[b])

        g = [None] * _NBUF
        w = [None] * _NBUF
        for c in range(_DEPTH):
            load_idx(c, c % _NBUF)
            g[c % _NBUF] = start_gather(c % _NBUF)
        for c in range(nchunk):
            b = c % _NBUF
            g[b].wait()
            g[b] = None
            w[b] = start_write(c, b)
            n = c + _DEPTH
            if n < nchunk:
                nb = n % _NBUF
                if w[nb] is not None:
                    w[nb].wait()
                    w[nb] = None
                load_idx(n, nb)
                g[nb] = start_gather(nb)
        for b in range(_NBUF):
            if w[b] is not None:
                w[b].wait()

    return gather_k(idx, table)


def _prologue_body(coords, t, Wf, te_W, te_b, tm_W, tm_b, in_W, in_b, bc,
                   h0, tpn_all, elem):
    xp = (2.0 * jnp.pi) * t[...] * Wf[...]
    tf = jnp.concatenate([jnp.sin(xp), jnp.cos(xp)], axis=-1)
    tf = _mm(tf, te_W[...]) + te_b[...]
    tf = tf * jax.nn.sigmoid(tf)
    for l in range(NL):
        tpn_all[l * B:(l + 1) * B, :] = _mm(tf, tm_W[l]) + tm_b[l]
    h0[...] = _mm(coords[...], in_W[...]) + in_b[...]
    gid = lax.broadcasted_iota(jnp.int32, (B, N), 0)
    cntg = jnp.sum((gid == bc[...]).astype(jnp.float32), axis=1,
                   keepdims=True)
    elem[...] = jnp.maximum(cntg * HID, 1.0)


def _prologue(coords, batch, t, Wf, te_W, te_b, tm_W, tm_b, in_W, in_b):
    return pl.pallas_call(
        _prologue_body,
        out_shape=[jax.ShapeDtypeStruct((N, HID), jnp.float32),
                   jax.ShapeDtypeStruct((NL * B, HID), jnp.float32),
                   jax.ShapeDtypeStruct((B, 1), jnp.float32)],
    )(coords, t.reshape(B, 1), Wf.reshape(1, TED // 2), te_W,
      te_b.reshape(1, TED), tm_W, tm_b, in_W, in_b.reshape(1, HID),
      batch.reshape(1, N))


def _tables_body(h, br, tpn_g, Wd, Ws, pre_b,
                 x_out, yd_out, ys_out):
    oh = (br[...] == lax.broadcasted_iota(jnp.int32, (N, B), 1)).astype(
        jnp.float32)
    x = h[...] + _mm_exact(oh, tpn_g[...])
    x_out[...] = x
    yd_out[...] = _mm(x, Wd[...]) + pre_b[...]
    ys_out[...] = _mm(x, Ws[...])


def _layer_tables(h, batch_r, tpn_g, Wd, Ws, pre_bl):
    return pl.pallas_call(
        _tables_body,
        out_shape=[jax.ShapeDtypeStruct((N, HID), jnp.float32),
                   jax.ShapeDtypeStruct((N, HID), jnp.float32),
                   jax.ShapeDtypeStruct((N, HID), jnp.float32)],
    )(h, batch_r, tpn_g, Wd, Ws, pre_bl.reshape(1, HID))


def _agg_body(x, yd, G, elen, br, bc, ee_W, ee_b, We, Px, P1, P2, P3,
              post_b, lin_W, lin_b, hn_out, gstat_out):
    centers = (lax.broadcasted_iota(jnp.int32, (1, NBASIS), 1).astype(
        jnp.float32) + 1.0) * _STEP
    Gb = G[...]
    ydv = yd[...]
    elen_v = elen[...]
    basis_list = []
    for j in range(MAXNB):
        elen_j = elen_v[:, j:j + 1]
        diff = (elen_j - centers) * (1.0 / _STEP)
        basis_list.append(jnp.exp(-(diff * diff)) * (1.0 / 1.12))
    basis_all = jnp.concatenate(basis_list, axis=0)
    e_all = _mm(basis_all, ee_W[...]) + ee_b[...]
    E3 = _mm(e_all, We[...]).reshape(MAXNB, RB, HID)
    s1 = jnp.zeros((RB, HID), jnp.float32)
    s2 = jnp.zeros((RB, HID), jnp.float32)
    mx = jnp.full((RB, HID), -jnp.inf, jnp.float32)
    mn = jnp.full((RB, HID), jnp.inf, jnp.float32)
    cnt = jnp.zeros((RB, 1), jnp.float32)
    for j in range(MAXNB):
        elen_j = elen_v[:, j:j + 1]
        m_j = ydv + Gb[j] + E3[j]
        ok = elen_j <= RADIUS
        w_j = ok.astype(jnp.float32)
        s1 = s1 + w_j * m_j
        s2 = s2 + w_j * (m_j * m_j)
        mx = jnp.maximum(mx, jnp.where(ok, m_j, -jnp.inf))
        mn = jnp.minimum(mn, jnp.where(ok, m_j, jnp.inf))
        cnt = cnt + w_j
    cntc = jnp.maximum(cnt, 1.0)
    inv = 1.0 / cntc
    mean = s1 * inv
    mean2 = s2 * inv
    std = jnp.sqrt(jax.nn.relu(mean2 - mean * mean) + 1e-5)
    has = cnt > 0.0
    mx = jnp.where(has, mx, 0.0)
    mn = jnp.where(has, mn, 0.0)
    agg = jnp.concatenate([mean, mx, mn, std], axis=-1)
    dlog = jnp.log(cntc + 1.0)
    f1 = dlog * (1.0 / AVG_DEG_LOG)
    f2 = AVG_DEG_LOG / dlog
    out = (_mm(x[...], Px[...]) + _mm(agg, P1[...]) + _mm(f1 * agg, P2[...])
           + _mm(f2 * agg, P3[...]) + post_b[...])
    hn = _mm(out, lin_W[...]) + lin_b[...]
    hn_out[...] = hn
    ohT = (lax.broadcasted_iota(jnp.int32, (B, RB), 0) == bc[...]).astype(
        jnp.float32)
    rows = jnp.concatenate([jnp.sum(hn, axis=1, keepdims=True),
                            jnp.sum(hn * hn, axis=1, keepdims=True)], axis=1)
    gstat_out[...] = _mm_exact(ohT, rows).reshape(1, B, 2)


def _layer_agg(x, yd, G, elen, batch_r, batch_c, ee_Wl, ee_bl, We,
               Px, P1, P2, P3, post_bl, lin_Wl, lin_bl):
    nn = x.shape[0]
    nblk = nn // RB
    blk = lambda r, c: pl.BlockSpec((r, c), lambda i: (i, 0))
    full = lambda r, c: pl.BlockSpec((r, c), lambda i: (0, 0))
    return pl.pallas_call(
        _agg_body,
        grid=(nblk,),
        in_specs=[blk(RB, HID), blk(RB, HID),
                  pl.BlockSpec((SLOTS, RB, HID), lambda i: (0, i, 0)),
                  blk(RB, SLOTS), blk(RB, 1),
                  pl.BlockSpec((1, RB), lambda i: (0, i)),
                  full(NBASIS, HID), full(1, HID), full(HID, HID),
                  full(HID, HID),
                  full(4 * HID, HID), full(4 * HID, HID), full(4 * HID, HID),
                  full(1, HID), full(HID, HID), full(1, HID)],
        out_specs=[pl.BlockSpec((RB, HID), lambda i: (i, 0)),
                   pl.BlockSpec((1, B, 2), lambda i: (i, 0, 0))],
        out_shape=[jax.ShapeDtypeStruct((nn, HID), jnp.float32),
                   jax.ShapeDtypeStruct((nblk, B, 2), jnp.float32)],
    )(x, yd, G, elen, batch_r, batch_c, ee_Wl, ee_bl.reshape(1, HID), We,
      Px, P1, P2, P3, post_bl.reshape(1, HID), lin_Wl,
      lin_bl.reshape(1, HID))


def _graph_stats(gstat, elem):
    tot3 = jnp.sum(gstat[...], axis=0, keepdims=False)
    inv_elem = 1.0 / elem[...]
    gmean = tot3[:, 0:1] * inv_elem
    gvar = tot3[:, 1:2] * inv_elem - gmean * gmean
    rstd = 1.0 / jnp.sqrt(gvar + 1e-5)
    return jnp.concatenate([gmean, rstd], axis=1)


def _norm_tables_body(hn, hres, br, gstat, elem, lnw, lnb, tpn_g, Wd, Ws,
                      pre_b, h_out, x_out, yd_out, ys_out):
    stats = _graph_stats(gstat, elem)
    oh = (br[...] == lax.broadcasted_iota(jnp.int32, (N, B), 1)).astype(
        jnp.float32)
    nst = _mm_exact(oh, stats)
    hnorm = (hn[...] - nst[:, 0:1]) * nst[:, 1:2] * lnw[...] + lnb[...]
    a = hnorm + hres[...]
    h = a * jax.nn.sigmoid(a)
    h_out[...] = h
    x = h + _mm_exact(oh, tpn_g[...])
    x_out[...] = x
    yd_out[...] = _mm(x, Wd[...]) + pre_b[...]
    ys_out[...] = _mm(x, Ws[...])


def _layer_norm_tables(hn, hres, batch_r, gstat, elem, lnw, lnb, tpn_g,
                       Wd, Ws, pre_bl):
    return pl.pallas_call(
        _norm_tables_body,
        out_shape=[jax.ShapeDtypeStruct((N, HID), jnp.float32),
                   jax.ShapeDtypeStruct((N, HID), jnp.float32),
                   jax.ShapeDtypeStruct((N, HID), jnp.float32),
                   jax.ShapeDtypeStruct((N, HID), jnp.float32)],
    )(hn, hres, batch_r, gstat, elem, lnw.reshape(1, HID),
      lnb.reshape(1, HID), tpn_g, Wd, Ws, pre_bl.reshape(1, HID))


def _norm_final_body(hn, hres, br, gstat, elem, lnw, lnb, out_W, out_b, o):
    stats = _graph_stats(gstat, elem)
    oh = (br[...] == lax.broadcasted_iota(jnp.int32, (N, B), 1)).astype(
        jnp.float32)
    nst = _mm_exact(oh, stats)
    hnorm = (hn[...] - nst[:, 0:1]) * nst[:, 1:2] * lnw[...] + lnb[...]
    a = hnorm + hres[...]
    h = a * jax.nn.sigmoid(a)
    o[...] = _mm(h, out_W[...]) + out_b[...]


def _layer_norm_final(hn, hres, batch_r, gstat, elem, lnw, lnb, out_W, out_b):
    return pl.pallas_call(
        _norm_final_body,
        out_shape=jax.ShapeDtypeStruct((N, 3), jnp.float32),
    )(hn, hres, batch_r, gstat, elem, lnw.reshape(1, HID),
      lnb.reshape(1, HID), out_W, out_b.reshape(1, 3))


def _norm_body(hn, hres, br, gstat, elem, lnw, lnb, h_out):
    tot3 = jnp.sum(gstat[...], axis=0, keepdims=False)
    tot = tot3[:, 0:1]
    tot2 = tot3[:, 1:2]
    inv_elem = 1.0 / elem[...]
    gmean = tot * inv_elem
    gvar = tot2 * inv_elem - gmean * gmean
    rstd = 1.0 / jnp.sqrt(gvar + 1e-5)
    stats = jnp.concatenate([gmean, rstd], axis=1)
    oh = (br[...] == lax.broadcasted_iota(jnp.int32, (RB, B), 1)).astype(
        jnp.float32)
    nst = _mm_exact(oh, stats)
    nmean = nst[:, 0:1]
    nrstd = nst[:, 1:2]
    hnorm = (hn[...] - nmean) * nrstd * lnw[...] + lnb[...]
    a = hnorm + hres[...]
    h_out[...] = a * jax.nn.sigmoid(a)


def _layer_norm(hn, hres, batch_r, gstat, elem, lnw, lnb):
    blk = lambda r, c: pl.BlockSpec((r, c), lambda i: (i, 0))
    full = lambda r, c: pl.BlockSpec((r, c), lambda i: (0, 0))
    return pl.pallas_call(
        _norm_body,
        grid=(NBLK,),
        in_specs=[blk(RB, HID), blk(RB, HID), blk(RB, 1),
                  pl.BlockSpec((NBLK, B, 2), lambda i: (0, 0, 0)),
                  full(B, 1), full(1, HID), full(1, HID)],
        out_specs=pl.BlockSpec((RB, HID), lambda i: (i, 0)),
        out_shape=jax.ShapeDtypeStruct((N, HID), jnp.float32),
    )(hn, hres, batch_r, gstat, elem, lnw.reshape(1, HID),
      lnb.reshape(1, HID))


def _final_body(h, out_W, out_b, o):
    o[...] = _mm(h[...], out_W[...]) + out_b[...]


def _final(h, out_W, out_b):
    return pl.pallas_call(
        _final_body,
        out_shape=jax.ShapeDtypeStruct((N, 3), jnp.float32),
    )(h, out_W, out_b.reshape(1, 3))


def kernel(coords, batch, t, Wf, te_W, te_b, in_W, in_b, out_W, out_b,
           tm_W, tm_b, ee_W, ee_b, pre_W, pre_b, post_W, post_b,
           lin_W, lin_b, ln_w, ln_b):
    batch = batch.astype(jnp.int32)
    batch_r = batch.reshape(N, 1)
    batch_c = batch.reshape(1, N)

    src, elen = _neighbor_search(coords, batch)
    NH = N // 2
    idx_half = [src[:NH].T.reshape(NH * SLOTS),
                src[NH:].T.reshape(NH * SLOTS)]

    h0, tpn_all, elem = _prologue(coords, batch, t, Wf, te_W, te_b,
                                  tm_W, tm_b, in_W, in_b)
    h = h0
    Wd = [pre_W[l, 0 * HID:1 * HID] for l in range(NL)]
    Ws = [pre_W[l, 1 * HID:2 * HID] for l in range(NL)]
    We = [pre_W[l, 2 * HID:3 * HID] for l in range(NL)]
    x, yd, ys = _layer_tables(h0, batch_r, tpn_all[0:B], Wd[0], Ws[0],
                              pre_b[0])
    for l in range(NL):
        G1 = _sc_gather(ys, idx_half[0]).reshape(SLOTS, NH, HID)
        G2 = _sc_gather(ys, idx_half[1]).reshape(SLOTS, NH, HID)
        Px = post_W[l, 0:HID]
        P1 = post_W[l, HID + 0 * 4 * HID:HID + 1 * 4 * HID]
        P2 = post_W[l, HID + 1 * 4 * HID:HID + 2 * 4 * HID]
        P3 = post_W[l, HID + 2 * 4 * HID:HID + 3 * 4 * HID]
        hn1, gstat1 = _layer_agg(
            x[:NH], yd[:NH], G1, elen[:NH], batch_r[:NH],
            batch_c[:, :NH], ee_W[l], ee_b[l], We[l], Px, P1, P2, P3,
            post_b[l], lin_W[l], lin_b[l])
        hn2, gstat2 = _layer_agg(
            x[NH:], yd[NH:], G2, elen[NH:], batch_r[NH:],
            batch_c[:, NH:], ee_W[l], ee_b[l], We[l], Px, P1, P2, P3,
            post_b[l], lin_W[l], lin_b[l])
        hn = jnp.concatenate([hn1, hn2], axis=0)
        gstat = jnp.concatenate([gstat1, gstat2], axis=0)
        if l + 1 < NL:
            h, x, yd, ys = _layer_norm_tables(
                hn, h, batch_r, gstat, elem, ln_w[l], ln_b[l],
                tpn_all[(l + 1) * B:(l + 2) * B], Wd[l + 1], Ws[l + 1],
                pre_b[l + 1])
        else:
            return _layer_norm_final(hn, h, batch_r, gstat, elem,
                                     ln_w[l], ln_b[l], out_W, out_b)


# speedup vs baseline: 1.2241x; 1.0004x over previous
import functools

import jax
import jax.numpy as jnp
import numpy as np
from jax import lax
from jax.experimental import pallas as pl
from jax.experimental.pallas import tpu as pltpu
from jax.experimental.pallas import tpu_sc as plsc

N = 4096
B = 16
HID = 128
TED = 128
NL = 5
RADIUS = 1.5
MAXNB = 30
SLOTS = 32
NBASIS = 16
AVG_DEG_LOG = float(np.log(31.0))
RB = 256
NBLK = N // RB
E = N * SLOTS

_vals = np.linspace(0.0, RADIUS, NBASIS + 2)
_STEP = float(_vals[1] - _vals[0])


def _mm(a, b):
    return lax.dot(a.astype(jnp.bfloat16), b.astype(jnp.bfloat16),
                   preferred_element_type=jnp.float32)


def _mm_exact(a, b):
    return lax.dot(a, b, precision=lax.Precision.HIGHEST)


def _nbr_body(cxr, cyr, czr, br, cxc, cyc, czc, bc, src_out, elen_out, d_s):
    i = pl.program_id(0)
    dx = cxr[...] - cxc[...]
    dy = cyr[...] - cyc[...]
    dz = czr[...] - czc[...]
    d = jnp.sqrt((dx * dx + dy * dy) + dz * dz)
    rowid = i * RB + lax.broadcasted_iota(jnp.int32, (RB, N), 0)
    colid = lax.broadcasted_iota(jnp.int32, (RB, N), 1)
    invalid = (br[...] != bc[...]) | (rowid == colid)
    d_s[...] = jnp.where(invalid, jnp.inf, d)

    slot = lax.broadcasted_iota(jnp.int32, (RB, SLOTS), 1)

    def body(k, carry):
        vals, idxs = carry
        dcur = d_s[...]
        rowmin = jnp.min(dcur, axis=1, keepdims=True)
        amin = jnp.min(jnp.where(dcur == rowmin, colid, N), axis=1,
                       keepdims=True)
        d_s[...] = jnp.where(colid == amin, jnp.inf, dcur)
        hit = slot == k
        vals = jnp.where(hit, rowmin, vals)
        idxs = jnp.where(hit, amin, idxs)
        return vals, idxs

    vals0 = jnp.full((RB, SLOTS), jnp.inf, jnp.float32)
    idxs0 = jnp.zeros((RB, SLOTS), jnp.int32)
    vals, idxs = lax.fori_loop(0, MAXNB, body, (vals0, idxs0))
    elen_out[...] = vals
    src_out[...] = idxs


def _neighbor_search(coords, batch):
    cxr = coords[:, 0:1]
    cyr = coords[:, 1:2]
    czr = coords[:, 2:3]
    cxc = coords[:, 0].reshape(1, N)
    cyc = coords[:, 1].reshape(1, N)
    czc = coords[:, 2].reshape(1, N)
    br = batch.reshape(N, 1)
    bc = batch.reshape(1, N)
    row_spec = pl.BlockSpec((RB, 1), lambda i: (i, 0))
    col_spec = pl.BlockSpec((1, N), lambda i: (0, 0))
    return pl.pallas_call(
        _nbr_body,
        grid=(NBLK,),
        in_specs=[row_spec, row_spec, row_spec, row_spec,
                  col_spec, col_spec, col_spec, col_spec],
        out_specs=[pl.BlockSpec((RB, SLOTS), lambda i: (i, 0)),
                   pl.BlockSpec((RB, SLOTS), lambda i: (i, 0))],
        out_shape=[jax.ShapeDtypeStruct((N, SLOTS), jnp.int32),
                   jax.ShapeDtypeStruct((N, SLOTS), jnp.float32)],
        scratch_shapes=[pltpu.VMEM((RB, N), jnp.float32)],
    )(cxr, cyr, czr, br, cxc, cyc, czc, bc)


_NW = 32
_BPW = E // _NW
_CH = 128
_NCHUNK = _BPW // _CH
_NBUF = 4
_DEPTH = 3


def _sc_gather(table, idx):
    mesh = plsc.VectorSubcoreMesh(core_axis_name="c", subcore_axis_name="s")
    n_rows = idx.shape[0]
    bpw = n_rows // _NW
    nchunk = bpw // _CH

    @functools.partial(
        pl.kernel,
        out_type=jax.ShapeDtypeStruct((n_rows, HID), jnp.float32),
        mesh=mesh,
        scratch_types=(
            [pltpu.VMEM((_CH,), jnp.int32) for _ in range(_NBUF)]
            + [pltpu.VMEM((_CH, HID), jnp.float32) for _ in range(_NBUF)]
            + [pltpu.SemaphoreType.DMA for _ in range(2 * _NBUF)]
        ),
    )
    def gather_k(idx_hbm, table_hbm, out_hbm, *bufs):
        ivs = bufs[0:_NBUF]
        rvs = bufs[_NBUF:2 * _NBUF]
        gss = bufs[2 * _NBUF:3 * _NBUF]
        wss = bufs[3 * _NBUF:4 * _NBUF]
        wid = lax.axis_index("s") * 2 + lax.axis_index("c")
        base = wid * bpw

        def load_idx(c, b):
            pltpu.sync_copy(idx_hbm.at[pl.ds(base + c * _CH, _CH)], ivs[b])

        def start_gather(b):
            return pltpu.async_copy(table_hbm.at[ivs[b]], rvs[b], gss[b])

        def start_write(c, b):
            return pltpu.async_copy(
                rvs[b], out_hbm.at[pl.ds(base + c * _CH, _CH)], wss[b])

        g = [None] * _NBUF
        w = [None] * _NBUF
        for c in range(_DEPTH):
            load_idx(c, c % _NBUF)
            g[c % _NBUF] = start_gather(c % _NBUF)
        for c in range(nchunk):
            b = c % _NBUF
            g[b].wait()
            g[b] = None
            w[b] = start_write(c, b)
            n = c + _DEPTH
            if n < nchunk:
                nb = n % _NBUF
                if w[nb] is not None:
                    w[nb].wait()
                    w[nb] = None
                load_idx(n, nb)
                g[nb] = start_gather(nb)
        for b in range(_NBUF):
            if w[b] is not None:
                w[b].wait()

    return gather_k(idx, table)


def _prologue_body(coords, t, Wf, te_W, te_b, tm_W, tm_b, in_W, in_b, bc,
                   h0, tpn_all, elem):
    xp = (2.0 * jnp.pi) * t[...] * Wf[...]
    tf = jnp.concatenate([jnp.sin(xp), jnp.cos(xp)], axis=-1)
    tf = _mm(tf, te_W[...]) + te_b[...]
    tf = tf * jax.nn.sigmoid(tf)
    for l in range(NL):
        tpn_all[l * B:(l + 1) * B, :] = _mm(tf, tm_W[l]) + tm_b[l]
    h0[...] = _mm(coords[...], in_W[...]) + in_b[...]
    gid = lax.broadcasted_iota(jnp.int32, (B, N), 0)
    cntg = jnp.sum((gid == bc[...]).astype(jnp.float32), axis=1,
                   keepdims=True)
    elem[...] = jnp.maximum(cntg * HID, 1.0)


def _prologue(coords, batch, t, Wf, te_W, te_b, tm_W, tm_b, in_W, in_b):
    return pl.pallas_call(
        _prologue_body,
        out_shape=[jax.ShapeDtypeStruct((N, HID), jnp.float32),
                   jax.ShapeDtypeStruct((NL * B, HID), jnp.float32),
                   jax.ShapeDtypeStruct((B, 1), jnp.float32)],
    )(coords, t.reshape(B, 1), Wf.reshape(1, TED // 2), te_W,
      te_b.reshape(1, TED), tm_W, tm_b, in_W, in_b.reshape(1, HID),
      batch.reshape(1, N))


def _tables_body(h, br, tpn_g, Wd, Ws, pre_b,
                 x_out, yd_out, ys_out):
    oh = (br[...] == lax.broadcasted_iota(jnp.int32, (N, B), 1)).astype(
        jnp.float32)
    x = h[...] + _mm_exact(oh, tpn_g[...])
    x_out[...] = x
    yd_out[...] = _mm(x, Wd[...]) + pre_b[...]
    ys_out[...] = _mm(x, Ws[...])


def _layer_tables(h, batch_r, tpn_g, Wd, Ws, pre_bl):
    return pl.pallas_call(
        _tables_body,
        out_shape=[jax.ShapeDtypeStruct((N, HID), jnp.float32),
                   jax.ShapeDtypeStruct((N, HID), jnp.float32),
                   jax.ShapeDtypeStruct((N, HID), jnp.float32)],
    )(h, batch_r, tpn_g, Wd, Ws, pre_bl.reshape(1, HID))


def _agg_body(x, yd, G, elen, br, bc, ee_W, ee_b, We, Px, P1, P2, P3,
              post_b, lin_W, lin_b, hn_out, gstat_out):
    centers = (lax.broadcasted_iota(jnp.int32, (1, NBASIS), 1).astype(
        jnp.float32) + 1.0) * _STEP
    Gb = G[...]
    ydv = yd[...]
    elen_v = elen[...]
    basis_list = []
    for j in range(MAXNB):
        elen_j = elen_v[:, j:j + 1]
        diff = (elen_j - centers) * (1.0 / _STEP)
        basis_list.append(jnp.exp(-(diff * diff)) * (1.0 / 1.12))
    basis_all = jnp.concatenate(basis_list, axis=0)
    e_all = _mm(basis_all, ee_W[...]) + ee_b[...]
    E3 = _mm(e_all, We[...]).reshape(MAXNB, RB, HID)
    s1 = jnp.zeros((RB, HID), jnp.float32)
    s2 = jnp.zeros((RB, HID), jnp.float32)
    mx = jnp.full((RB, HID), -jnp.inf, jnp.float32)
    mn = jnp.full((RB, HID), jnp.inf, jnp.float32)
    cnt = jnp.zeros((RB, 1), jnp.float32)
    for j in range(MAXNB):
        elen_j = elen_v[:, j:j + 1]
        m_j = ydv + Gb[j] + E3[j]
        ok = elen_j <= RADIUS
        w_j = ok.astype(jnp.float32)
        s1 = s1 + w_j * m_j
        s2 = s2 + w_j * (m_j * m_j)
        mx = jnp.maximum(mx, jnp.where(ok, m_j, -jnp.inf))
        mn = jnp.minimum(mn, jnp.where(ok, m_j, jnp.inf))
        cnt = cnt + w_j
    cntc = jnp.maximum(cnt, 1.0)
    inv = 1.0 / cntc
    mean = s1 * inv
    mean2 = s2 * inv
    std = jnp.sqrt(jax.nn.relu(mean2 - mean * mean) + 1e-5)
    has = cnt > 0.0
    mx = jnp.where(has, mx, 0.0)
    mn = jnp.where(has, mn, 0.0)
    agg = jnp.concatenate([mean, mx, mn, std], axis=-1)
    dlog = jnp.log(cntc + 1.0)
    f1 = dlog * (1.0 / AVG_DEG_LOG)
    f2 = AVG_DEG_LOG / dlog
    out = (_mm(x[...], Px[...]) + _mm(agg, P1[...]) + _mm(f1 * agg, P2[...])
           + _mm(f2 * agg, P3[...]) + post_b[...])
    hn = _mm(out, lin_W[...]) + lin_b[...]
    hn_out[...] = hn
    ohT = (lax.broadcasted_iota(jnp.int32, (B, RB), 0) == bc[...]).astype(
        jnp.float32)
    rows = jnp.concatenate([jnp.sum(hn, axis=1, keepdims=True),
                            jnp.sum(hn * hn, axis=1, keepdims=True)], axis=1)
    gstat_out[...] = _mm_exact(ohT, rows).reshape(1, B, 2)


def _layer_agg(x, yd, G, elen, batch_r, batch_c, ee_Wl, ee_bl, We,
               Px, P1, P2, P3, post_bl, lin_Wl, lin_bl):
    nn = x.shape[0]
    nblk = nn // RB
    blk = lambda r, c: pl.BlockSpec((r, c), lambda i: (i, 0))
    full = lambda r, c: pl.BlockSpec((r, c), lambda i: (0, 0))
    return pl.pallas_call(
        _agg_body,
        grid=(nblk,),
        in_specs=[blk(RB, HID), blk(RB, HID),
                  pl.BlockSpec((SLOTS, RB, HID), lambda i: (0, i, 0)),
                  blk(RB, SLOTS), blk(RB, 1),
                  pl.BlockSpec((1, RB), lambda i: (0, i)),
                  full(NBASIS, HID), full(1, HID), full(HID, HID),
                  full(HID, HID),
                  full(4 * HID, HID), full(4 * HID, HID), full(4 * HID, HID),
                  full(1, HID), full(HID, HID), full(1, HID)],
        out_specs=[pl.BlockSpec((RB, HID), lambda i: (i, 0)),
                   pl.BlockSpec((1, B, 2), lambda i: (i, 0, 0))],
        out_shape=[jax.ShapeDtypeStruct((nn, HID), jnp.float32),
                   jax.ShapeDtypeStruct((nblk, B, 2), jnp.float32)],
    )(x, yd, G, elen, batch_r, batch_c, ee_Wl, ee_bl.reshape(1, HID), We,
      Px, P1, P2, P3, post_bl.reshape(1, HID), lin_Wl,
      lin_bl.reshape(1, HID))


def _graph_stats(gstat, elem):
    tot3 = jnp.sum(gstat[...], axis=0, keepdims=False)
    inv_elem = 1.0 / elem[...]
    gmean = tot3[:, 0:1] * inv_elem
    gvar = tot3[:, 1:2] * inv_elem - gmean * gmean
    rstd = 1.0 / jnp.sqrt(gvar + 1e-5)
    return jnp.concatenate([gmean, rstd], axis=1)


def _norm_tables_body(hn, hres, br, gstat, elem, lnw, lnb, tpn_g, Wd, Ws,
                      pre_b, h_out, x_out, yd_out, ys_out):
    stats = _graph_stats(gstat, elem)
    oh = (br[...] == lax.broadcasted_iota(jnp.int32, (N, B), 1)).astype(
        jnp.float32)
    nst = _mm_exact(oh, stats)
    hnorm = (hn[...] - nst[:, 0:1]) * nst[:, 1:2] * lnw[...] + lnb[...]
    a = hnorm + hres[...]
    h = a * jax.nn.sigmoid(a)
    h_out[...] = h
    x = h + _mm_exact(oh, tpn_g[...])
    x_out[...] = x
    yd_out[...] = _mm(x, Wd[...]) + pre_b[...]
    ys_out[...] = _mm(x, Ws[...])


def _layer_norm_tables(hn, hres, batch_r, gstat, elem, lnw, lnb, tpn_g,
                       Wd, Ws, pre_bl):
    return pl.pallas_call(
        _norm_tables_body,
        out_shape=[jax.ShapeDtypeStruct((N, HID), jnp.float32),
                   jax.ShapeDtypeStruct((N, HID), jnp.float32),
                   jax.ShapeDtypeStruct((N, HID), jnp.float32),
                   jax.ShapeDtypeStruct((N, HID), jnp.float32)],
    )(hn, hres, batch_r, gstat, elem, lnw.reshape(1, HID),
      lnb.reshape(1, HID), tpn_g, Wd, Ws, pre_bl.reshape(1, HID))


def _norm_final_body(hn, hres, br, gstat, elem, lnw, lnb, out_W, out_b, o):
    stats = _graph_stats(gstat, elem)
    oh = (br[...] == lax.broadcasted_iota(jnp.int32, (N, B), 1)).astype(
        jnp.float32)
    nst = _mm_exact(oh, stats)
    hnorm = (hn[...] - nst[:, 0:1]) * nst[:, 1:2] * lnw[...] + lnb[...]
    a = hnorm + hres[...]
    h = a * jax.nn.sigmoid(a)
    o[...] = _mm(h, out_W[...]) + out_b[...]


def _layer_norm_final(hn, hres, batch_r, gstat, elem, lnw, lnb, out_W, out_b):
    return pl.pallas_call(
        _norm_final_body,
        out_shape=jax.ShapeDtypeStruct((N, 3), jnp.float32),
    )(hn, hres, batch_r, gstat, elem, lnw.reshape(1, HID),
      lnb.reshape(1, HID), out_W, out_b.reshape(1, 3))


def kernel(coords, batch, t, Wf, te_W, te_b, in_W, in_b, out_W, out_b,
           tm_W, tm_b, ee_W, ee_b, pre_W, pre_b, post_W, post_b,
           lin_W, lin_b, ln_w, ln_b):
    batch = batch.astype(jnp.int32)
    batch_r = batch.reshape(N, 1)
    batch_c = batch.reshape(1, N)

    src, elen = _neighbor_search(coords, batch)
    NH = N // 2
    idx_half = [src[:NH].T.reshape(NH * SLOTS),
                src[NH:].T.reshape(NH * SLOTS)]

    h0, tpn_all, elem = _prologue(coords, batch, t, Wf, te_W, te_b,
                                  tm_W, tm_b, in_W, in_b)
    h = h0
    Wd = [pre_W[l, 0 * HID:1 * HID] for l in range(NL)]
    Ws = [pre_W[l, 1 * HID:2 * HID] for l in range(NL)]
    We = [pre_W[l, 2 * HID:3 * HID] for l in range(NL)]
    x, yd, ys = _layer_tables(h0, batch_r, tpn_all[0:B], Wd[0], Ws[0],
                              pre_b[0])
    for l in range(NL):
        G1 = _sc_gather(ys, idx_half[0]).reshape(SLOTS, NH, HID)
        G2 = _sc_gather(ys, idx_half[1]).reshape(SLOTS, NH, HID)
        Px = post_W[l, 0:HID]
        P1 = post_W[l, HID + 0 * 4 * HID:HID + 1 * 4 * HID]
        P2 = post_W[l, HID + 1 * 4 * HID:HID + 2 * 4 * HID]
        P3 = post_W[l, HID + 2 * 4 * HID:HID + 3 * 4 * HID]
        hn1, gstat1 = _layer_agg(
            x[:NH], yd[:NH], G1, elen[:NH], batch_r[:NH],
            batch_c[:, :NH], ee_W[l], ee_b[l], We[l], Px, P1, P2, P3,
            post_b[l], lin_W[l], lin_b[l])
        hn2, gstat2 = _layer_agg(
            x[NH:], yd[NH:], G2, elen[NH:], batch_r[NH:],
            batch_c[:, NH:], ee_W[l], ee_b[l], We[l], Px, P1, P2, P3,
            post_b[l], lin_W[l], lin_b[l])
        hn = jnp.concatenate([hn1, hn2], axis=0)
        gstat = jnp.concatenate([gstat1, gstat2], axis=0)
        if l + 1 < NL:
            h, x, yd, ys = _layer_norm_tables(
                hn, h, batch_r, gstat, elem, ln_w[l], ln_b[l],
                tpn_all[(l + 1) * B:(l + 2) * B], Wd[l + 1], Ws[l + 1],
                pre_b[l + 1])
        else:
            return _layer_norm_final(hn, h, batch_r, gstat, elem,
                                     ln_w[l], ln_b[l], out_W, out_b)
